# baseline (speedup 1.0000x reference)
import functools

import jax
import jax.numpy as jnp
from jax import lax
from jax.experimental import pallas as pl
from jax.experimental.pallas import tpu as pltpu
from jax.experimental.pallas import tpu_sc as plsc

N = 10000
E = 320000
D = 128
EPS = 1e-7
LN_EPS = 1e-5

RB = 1000
GRID = N // RB

SC_TILES = 16
IDX_B = 128
CHUNKS = 160
EDGES_PER_TILE = CHUNKS * IDX_B
E_PAD = SC_TILES * EDGES_PER_TILE
NP = 10112
RPT = NP // SC_TILES


def _fc_body(x_ref, w_ref, b_ref, h_ref, k_ref):
    h = jnp.dot(x_ref[...], w_ref[...], preferred_element_type=jnp.float32)
    h = jnp.maximum(h + b_ref[...], 0.0)
    h_ref[...] = h
    m = jnp.max(h, axis=0, keepdims=True) + EPS
    @pl.when(pl.program_id(0) == 0)
    def _():
        k_ref[...] = m
    k_ref[...] = jnp.maximum(k_ref[...], m)


def _fc_call(x, w, b):
    return pl.pallas_call(
        _fc_body,
        grid=(GRID,),
        in_specs=[
            pl.BlockSpec((RB, D), lambda i: (i, 0)),
            pl.BlockSpec((D, D), lambda i: (0, 0)),
            pl.BlockSpec((1, D), lambda i: (0, 0)),
        ],
        out_specs=[
            pl.BlockSpec((RB, D), lambda i: (i, 0)),
            pl.BlockSpec((1, D), lambda i: (0, 0)),
        ],
        out_shape=[
            jax.ShapeDtypeStruct((N, D), jnp.float32),
            jax.ShapeDtypeStruct((1, D), jnp.float32),
        ],
    )(x, w, b)


def _prep_body(h_ref, k_ref, t_ref, u_ref, v_ref):
    r = jnp.maximum(h_ref[...], 0.0) + EPS
    t = t_ref[0, 0]
    w = jnp.exp(t * (r - k_ref[...]))
    u_ref[...] = r * w
    v_ref[...] = w


def _prep_call(h, k, t):
    return pl.pallas_call(
        _prep_body,
        grid=(GRID,),
        in_specs=[
            pl.BlockSpec((RB, D), lambda i: (i, 0)),
            pl.BlockSpec((1, D), lambda i: (0, 0)),
            pl.BlockSpec((1, 1), lambda i: (0, 0)),
        ],
        out_specs=[
            pl.BlockSpec((RB, D), lambda i: (i, 0)),
            pl.BlockSpec((RB, D), lambda i: (i, 0)),
        ],
        out_shape=[
            jax.ShapeDtypeStruct((N, D), jnp.float32),
            jax.ShapeDtypeStruct((N, D), jnp.float32),
        ],
    )(h, k, t)


def _agg_body(u_hbm, v_hbm, eidx_hbm, z_hbm,
              num_hbm, den_hbm,
              ibuf, rows, acc, gsem):
    c = lax.axis_index("c")
    s = lax.axis_index("s")
    r0 = s * RPT

    pltpu.sync_copy(z_hbm, rows)
    for kk in range(4):
        pltpu.sync_copy(rows, acc.at[pl.ds(r0 + kk * IDX_B, IDX_B)])
    pltpu.sync_copy(rows.at[pl.ds(0, RPT - 4 * IDX_B)],
                    acc.at[pl.ds(r0 + 4 * IDX_B, RPT - 4 * IDX_B)])
    plsc.subcore_barrier()

    def run(tab_hbm):
        def chunk(j, carry):
            pltpu.sync_copy(eidx_hbm.at[s, j], ibuf)
            pltpu.async_copy(tab_hbm.at[ibuf.at[0]], rows, gsem).wait()
            pltpu.sync_copy(rows, acc.at[ibuf.at[1]], add=True)
            return carry

        lax.fori_loop(0, CHUNKS, chunk, 0)

    @pl.when(c == 0)
    def _():
        run(u_hbm)

    @pl.when(c == 1)
    def _():
        run(v_hbm)

    plsc.subcore_barrier()

    @pl.when(c == 0)
    def _():
        pltpu.sync_copy(acc.at[pl.ds(r0, RPT)], num_hbm.at[pl.ds(r0, RPT)])

    @pl.when(c == 1)
    def _():
        pltpu.sync_copy(acc.at[pl.ds(r0, RPT)], den_hbm.at[pl.ds(r0, RPT)])


def _agg_call(u, v, eidx, zrows):
    fn = pl.kernel(
        _agg_body,
        out_type=[
            jax.ShapeDtypeStruct((NP, D), jnp.float32),
            jax.ShapeDtypeStruct((NP, D), jnp.float32),
        ],
        mesh=plsc.VectorSubcoreMesh(core_axis_name="c", subcore_axis_name="s"),
        scratch_types=[
            pltpu.VMEM((2, IDX_B), jnp.int32),
            pltpu.VMEM((IDX_B, D), jnp.float32),
            pltpu.VMEM_SHARED((NP, D), jnp.float32),
            pltpu.SemaphoreType.DMA,
        ],
    )
    return fn(u, v, eidx, zrows)


def _post_body(first, num_ref, den_ref, h_ref, w1_ref, b1_ref, g1_ref,
               bt1_ref, w2_ref, b2_ref, ng_ref, nb_ref, h_out, k_ref):
    h = h_ref[...]
    agg = num_ref[...] / (den_ref[...] + 1e-16) + h
    hm = jnp.dot(agg, w1_ref[...], preferred_element_type=jnp.float32)
    hm = hm + b1_ref[...]
    mu = jnp.mean(hm, axis=-1, keepdims=True)
    dv = hm - mu
    var = jnp.mean(dv * dv, axis=-1, keepdims=True)
    hm = dv * lax.rsqrt(var + LN_EPS) * g1_ref[...] + bt1_ref[...]
    hm = jnp.maximum(hm, 0.0)
    hc = jnp.dot(hm, w2_ref[...], preferred_element_type=jnp.float32)
    hc = hc + b2_ref[...]
    if first:
        hnew = hc
    else:
        mu2 = jnp.mean(hc, axis=-1, keepdims=True)
        dv2 = hc - mu2
        var2 = jnp.mean(dv2 * dv2, axis=-1, keepdims=True)
        hc = dv2 * lax.rsqrt(var2 + LN_EPS) * ng_ref[...] + nb_ref[...]
        hnew = h + jnp.maximum(hc, 0.0)
    h_out[...] = hnew
    m = jnp.max(jnp.maximum(hnew, 0.0), axis=0, keepdims=True) + EPS
    @pl.when(pl.program_id(0) == 0)
    def _():
        k_ref[...] = m
    k_ref[...] = jnp.maximum(k_ref[...], m)


def _post_call(first, nump, denp, h, w1, b1, g1, bt1, w2, b2, ng, nb):
    return pl.pallas_call(
        functools.partial(_post_body, first),
        grid=(GRID,),
        in_specs=[
            pl.BlockSpec((RB, D), lambda i: (i, 0)),
            pl.BlockSpec((RB, D), lambda i: (i, 0)),
            pl.BlockSpec((RB, D), lambda i: (i, 0)),
            pl.BlockSpec((D, 2 * D), lambda i: (0, 0)),
            pl.BlockSpec((1, 2 * D), lambda i: (0, 0)),
            pl.BlockSpec((1, 2 * D), lambda i: (0, 0)),
            pl.BlockSpec((1, 2 * D), lambda i: (0, 0)),
            pl.BlockSpec((2 * D, D), lambda i: (0, 0)),
            pl.BlockSpec((1, D), lambda i: (0, 0)),
            pl.BlockSpec((1, D), lambda i: (0, 0)),
            pl.BlockSpec((1, D), lambda i: (0, 0)),
        ],
        out_specs=[
            pl.BlockSpec((RB, D), lambda i: (i, 0)),
            pl.BlockSpec((1, D), lambda i: (0, 0)),
        ],
        out_shape=[
            jax.ShapeDtypeStruct((N, D), jnp.float32),
            jax.ShapeDtypeStruct((1, D), jnp.float32),
        ],
    )(nump, denp, h, w1, b1, g1, bt1, w2, b2, ng, nb)


def _pool_body(h0_ref, h1_ref, h2_ref, h3_ref, wp0_ref, wp1_ref, wp2_ref,
               wp3_ref, bp_ref, wa_ref, ba_ref, wb_ref, bb_ref, wc_ref,
               bc_ref, out_ref, m_ref, d_ref, acc_ref):
    hp = jnp.dot(h0_ref[...], wp0_ref[...], preferred_element_type=jnp.float32)
    hp = hp + jnp.dot(h1_ref[...], wp1_ref[...], preferred_element_type=jnp.float32)
    hp = hp + jnp.dot(h2_ref[...], wp2_ref[...], preferred_element_type=jnp.float32)
    hp = hp + jnp.dot(h3_ref[...], wp3_ref[...], preferred_element_type=jnp.float32)
    hp = jnp.maximum(hp + bp_ref[...], 0.0)
    a = jnp.tanh(jnp.dot(hp, wa_ref[...], preferred_element_type=jnp.float32) + ba_ref[...])
    g = jnp.dot(hp, wb_ref[...], preferred_element_type=jnp.float32) + bb_ref[...]
    g = 1.0 / (1.0 + jnp.exp(-g))
    s = jnp.dot(a * g, wc_ref[...], preferred_element_type=jnp.float32) + bc_ref[...]

    @pl.when(pl.program_id(0) == 0)
    def _():
        m_ref[0, 0] = -1e30
        d_ref[0, 0] = 0.0
        acc_ref[...] = jnp.zeros_like(acc_ref)

    bm = jnp.max(s)
    m_old = m_ref[0, 0]
    m_new = jnp.maximum(m_old, bm)
    scale = jnp.exp(m_old - m_new)
    w = jnp.exp(s - m_new)
    d_ref[0, 0] = d_ref[0, 0] * scale + jnp.sum(w)
    acc_ref[...] = acc_ref[...] * scale + jnp.sum(w * hp, axis=0, keepdims=True)
    m_ref[0, 0] = m_new

    @pl.when(pl.program_id(0) == pl.num_programs(0) - 1)
    def _():
        out_ref[...] = acc_ref[...] / d_ref[0, 0]


def _pool_call(h0, h1, h2, h3, wp0, wp1, wp2, wp3, bp, wa, ba, wb, bb, wc, bc):
    blk = pl.BlockSpec((RB, D), lambda i: (i, 0))
    wblk = pl.BlockSpec((D, D), lambda i: (0, 0))
    vblk = pl.BlockSpec((1, D), lambda i: (0, 0))
    return pl.pallas_call(
        _pool_body,
        grid=(GRID,),
        in_specs=[blk, blk, blk, blk, wblk, wblk, wblk, wblk, vblk,
                  wblk, vblk, wblk, vblk,
                  pl.BlockSpec((D, 1), lambda i: (0, 0)),
                  pl.BlockSpec((1, 1), lambda i: (0, 0))],
        out_specs=pl.BlockSpec((1, D), lambda i: (0, 0)),
        out_shape=jax.ShapeDtypeStruct((1, D), jnp.float32),
        scratch_shapes=[
            pltpu.SMEM((1, 1), jnp.float32),
            pltpu.SMEM((1, 1), jnp.float32),
            pltpu.VMEM((1, D), jnp.float32),
        ],
    )(h0, h1, h2, h3, wp0, wp1, wp2, wp3, bp, wa, ba, wb, bb, wc, bc)


def kernel(x, edge_index, W_fc, b_fc, conv_W1, conv_b1, conv_ln_g,
           conv_ln_b, conv_W2, conv_b2, conv_t, norm_g, norm_b, W_phi,
           b_phi, Wa, ba, Wb, bb, Wc, bc):
    src = edge_index[0]
    dst = edge_index[1]
    pad = E_PAD - E
    srcp = jnp.concatenate([src, jnp.zeros((pad,), jnp.int32)])
    dstp = jnp.concatenate([dst, jnp.full((pad,), N, jnp.int32)])
    eidx = (jnp.stack([srcp, dstp])
            .reshape(2, SC_TILES, CHUNKS, IDX_B)
            .transpose(1, 2, 0, 3))
    zrows = jnp.zeros((IDX_B, D), jnp.float32)

    h, k = _fc_call(x, W_fc, b_fc.reshape(1, D))
    hs = [h]
    for i in range(3):
        u, v = _prep_call(h, k, conv_t[i].reshape(1, 1))
        nump, denp = _agg_call(u, v, eidx, zrows)
        h, k = _post_call(
            i == 0, nump, denp, h,
            conv_W1[i], conv_b1[i].reshape(1, 2 * D),
            conv_ln_g[i].reshape(1, 2 * D), conv_ln_b[i].reshape(1, 2 * D),
            conv_W2[i], conv_b2[i].reshape(1, D),
            norm_g[i].reshape(1, D), norm_b[i].reshape(1, D),
        )
        hs.append(h)

    H = _pool_call(
        hs[0], hs[1], hs[2], hs[3],
        W_phi[0:D], W_phi[D:2 * D], W_phi[2 * D:3 * D], W_phi[3 * D:4 * D],
        b_phi.reshape(1, D), Wa, ba.reshape(1, D), Wb, bb.reshape(1, D),
        Wc, bc.reshape(1, 1),
    )
    return H


# speedup vs baseline: 1.4780x; 1.4780x over previous
import functools

import jax
import jax.numpy as jnp
from jax import lax
from jax.experimental import pallas as pl
from jax.experimental.pallas import tpu as pltpu
from jax.experimental.pallas import tpu_sc as plsc

N = 10000
E = 320000
D = 128
EPS = 1e-7
LN_EPS = 1e-5

RB = 1000
GRID = N // RB

SC_TILES = 16
IDX_B = 128
CHUNKS = 157
EDGES_PER_TILE = CHUNKS * IDX_B
E_PAD = SC_TILES * EDGES_PER_TILE
NP = 10112
RPT = NP // SC_TILES


def _fc_body(x_ref, w_ref, b_ref, h_ref, k_ref):
    h = jnp.dot(x_ref[...], w_ref[...], preferred_element_type=jnp.float32)
    h = jnp.maximum(h + b_ref[...], 0.0)
    h_ref[...] = h
    m = jnp.max(h, axis=0, keepdims=True) + EPS
    @pl.when(pl.program_id(0) == 0)
    def _():
        k_ref[...] = m
    k_ref[...] = jnp.maximum(k_ref[...], m)


def _fc_call(x, w, b):
    return pl.pallas_call(
        _fc_body,
        grid=(GRID,),
        in_specs=[
            pl.BlockSpec((RB, D), lambda i: (i, 0)),
            pl.BlockSpec((D, D), lambda i: (0, 0)),
            pl.BlockSpec((1, D), lambda i: (0, 0)),
        ],
        out_specs=[
            pl.BlockSpec((RB, D), lambda i: (i, 0)),
            pl.BlockSpec((1, D), lambda i: (0, 0)),
        ],
        out_shape=[
            jax.ShapeDtypeStruct((N, D), jnp.float32),
            jax.ShapeDtypeStruct((1, D), jnp.float32),
        ],
    )(x, w, b)


def _prep_body(h_ref, k_ref, t_ref, u_ref, v_ref):
    r = jnp.maximum(h_ref[...], 0.0) + EPS
    t = t_ref[0, 0]
    w = jnp.exp(t * (r - k_ref[...]))
    u_ref[...] = r * w
    v_ref[...] = w


def _prep_call(h, k, t):
    return pl.pallas_call(
        _prep_body,
        grid=(GRID,),
        in_specs=[
            pl.BlockSpec((RB, D), lambda i: (i, 0)),
            pl.BlockSpec((1, D), lambda i: (0, 0)),
            pl.BlockSpec((1, 1), lambda i: (0, 0)),
        ],
        out_specs=[
            pl.BlockSpec((RB, D), lambda i: (i, 0)),
            pl.BlockSpec((RB, D), lambda i: (i, 0)),
        ],
        out_shape=[
            jax.ShapeDtypeStruct((N, D), jnp.float32),
            jax.ShapeDtypeStruct((N, D), jnp.float32),
        ],
    )(h, k, t)


def _agg_body(u_hbm, v_hbm, src_hbm, dst_hbm, z_hbm,
              num_hbm, den_hbm,
              sidx, didx, rows, acc, sem):
    c = lax.axis_index("c")
    s = lax.axis_index("s")
    r0 = s * RPT

    pltpu.sync_copy(z_hbm, rows)
    for kk in range(4):
        pltpu.sync_copy(rows, acc.at[pl.ds(r0 + kk * IDX_B, IDX_B)])
    pltpu.sync_copy(rows.at[pl.ds(0, RPT - 4 * IDX_B)],
                    acc.at[pl.ds(r0 + 4 * IDX_B, RPT - 4 * IDX_B)])
    plsc.subcore_barrier()

    def run(tab_hbm):
        ebase = s * EDGES_PER_TILE

        def chunk(j, carry):
            e0 = ebase + j * IDX_B
            pltpu.sync_copy(src_hbm.at[pl.ds(e0, IDX_B)], sidx)
            pltpu.sync_copy(dst_hbm.at[pl.ds(e0, IDX_B)], didx)
            pltpu.async_copy(tab_hbm.at[sidx], rows, sem).wait()
            pltpu.sync_copy(rows, acc.at[didx], add=True)
            return carry

        lax.fori_loop(0, CHUNKS, chunk, 0)

    @pl.when(c == 0)
    def _():
        run(u_hbm)

    @pl.when(c == 1)
    def _():
        run(v_hbm)

    plsc.subcore_barrier()

    @pl.when(c == 0)
    def _():
        pltpu.sync_copy(acc.at[pl.ds(r0, RPT)], num_hbm.at[pl.ds(r0, RPT)])

    @pl.when(c == 1)
    def _():
        pltpu.sync_copy(acc.at[pl.ds(r0, RPT)], den_hbm.at[pl.ds(r0, RPT)])


def _agg_call(u, v, srcp, dstp, zrows):
    fn = pl.kernel(
        _agg_body,
        out_type=[
            jax.ShapeDtypeStruct((NP, D), jnp.float32),
            jax.ShapeDtypeStruct((NP, D), jnp.float32),
        ],
        mesh=plsc.VectorSubcoreMesh(core_axis_name="c", subcore_axis_name="s"),
        scratch_types=[
            pltpu.VMEM((IDX_B,), jnp.int32),
            pltpu.VMEM((IDX_B,), jnp.int32),
            pltpu.VMEM((IDX_B, D), jnp.float32),
            pltpu.VMEM_SHARED((NP, D), jnp.float32),
            pltpu.SemaphoreType.DMA,
        ],
    )
    return fn(u, v, srcp, dstp, zrows)


def _post_body(first, num_ref, den_ref, h_ref, w1_ref, b1_ref, g1_ref,
               bt1_ref, w2_ref, b2_ref, ng_ref, nb_ref, h_out, k_ref):
    h = h_ref[...]
    agg = num_ref[...] / (den_ref[...] + 1e-16) + h
    hm = jnp.dot(agg, w1_ref[...], preferred_element_type=jnp.float32)
    hm = hm + b1_ref[...]
    mu = jnp.mean(hm, axis=-1, keepdims=True)
    dv = hm - mu
    var = jnp.mean(dv * dv, axis=-1, keepdims=True)
    hm = dv * lax.rsqrt(var + LN_EPS) * g1_ref[...] + bt1_ref[...]
    hm = jnp.maximum(hm, 0.0)
    hc = jnp.dot(hm, w2_ref[...], preferred_element_type=jnp.float32)
    hc = hc + b2_ref[...]
    if first:
        hnew = hc
    else:
        mu2 = jnp.mean(hc, axis=-1, keepdims=True)
        dv2 = hc - mu2
        var2 = jnp.mean(dv2 * dv2, axis=-1, keepdims=True)
        hc = dv2 * lax.rsqrt(var2 + LN_EPS) * ng_ref[...] + nb_ref[...]
        hnew = h + jnp.maximum(hc, 0.0)
    h_out[...] = hnew
    m = jnp.max(jnp.maximum(hnew, 0.0), axis=0, keepdims=True) + EPS
    @pl.when(pl.program_id(0) == 0)
    def _():
        k_ref[...] = m
    k_ref[...] = jnp.maximum(k_ref[...], m)


def _post_call(first, nump, denp, h, w1, b1, g1, bt1, w2, b2, ng, nb):
    return pl.pallas_call(
        functools.partial(_post_body, first),
        grid=(GRID,),
        in_specs=[
            pl.BlockSpec((RB, D), lambda i: (i, 0)),
            pl.BlockSpec((RB, D), lambda i: (i, 0)),
            pl.BlockSpec((RB, D), lambda i: (i, 0)),
            pl.BlockSpec((D, 2 * D), lambda i: (0, 0)),
            pl.BlockSpec((1, 2 * D), lambda i: (0, 0)),
            pl.BlockSpec((1, 2 * D), lambda i: (0, 0)),
            pl.BlockSpec((1, 2 * D), lambda i: (0, 0)),
            pl.BlockSpec((2 * D, D), lambda i: (0, 0)),
            pl.BlockSpec((1, D), lambda i: (0, 0)),
            pl.BlockSpec((1, D), lambda i: (0, 0)),
            pl.BlockSpec((1, D), lambda i: (0, 0)),
        ],
        out_specs=[
            pl.BlockSpec((RB, D), lambda i: (i, 0)),
            pl.BlockSpec((1, D), lambda i: (0, 0)),
        ],
        out_shape=[
            jax.ShapeDtypeStruct((N, D), jnp.float32),
            jax.ShapeDtypeStruct((1, D), jnp.float32),
        ],
    )(nump, denp, h, w1, b1, g1, bt1, w2, b2, ng, nb)


def _pool_body(h0_ref, h1_ref, h2_ref, h3_ref, wp0_ref, wp1_ref, wp2_ref,
               wp3_ref, bp_ref, wa_ref, ba_ref, wb_ref, bb_ref, wc_ref,
               bc_ref, out_ref, m_ref, d_ref, acc_ref):
    hp = jnp.dot(h0_ref[...], wp0_ref[...], preferred_element_type=jnp.float32)
    hp = hp + jnp.dot(h1_ref[...], wp1_ref[...], preferred_element_type=jnp.float32)
    hp = hp + jnp.dot(h2_ref[...], wp2_ref[...], preferred_element_type=jnp.float32)
    hp = hp + jnp.dot(h3_ref[...], wp3_ref[...], preferred_element_type=jnp.float32)
    hp = jnp.maximum(hp + bp_ref[...], 0.0)
    a = jnp.tanh(jnp.dot(hp, wa_ref[...], preferred_element_type=jnp.float32) + ba_ref[...])
    g = jnp.dot(hp, wb_ref[...], preferred_element_type=jnp.float32) + bb_ref[...]
    g = 1.0 / (1.0 + jnp.exp(-g))
    s = jnp.dot(a * g, wc_ref[...], preferred_element_type=jnp.float32) + bc_ref[...]

    @pl.when(pl.program_id(0) == 0)
    def _():
        m_ref[0, 0] = -1e30
        d_ref[0, 0] = 0.0
        acc_ref[...] = jnp.zeros_like(acc_ref)

    bm = jnp.max(s)
    m_old = m_ref[0, 0]
    m_new = jnp.maximum(m_old, bm)
    scale = jnp.exp(m_old - m_new)
    w = jnp.exp(s - m_new)
    d_ref[0, 0] = d_ref[0, 0] * scale + jnp.sum(w)
    acc_ref[...] = acc_ref[...] * scale + jnp.sum(w * hp, axis=0, keepdims=True)
    m_ref[0, 0] = m_new

    @pl.when(pl.program_id(0) == pl.num_programs(0) - 1)
    def _():
        out_ref[...] = acc_ref[...] / d_ref[0, 0]


def _pool_call(h0, h1, h2, h3, wp0, wp1, wp2, wp3, bp, wa, ba, wb, bb, wc, bc):
    blk = pl.BlockSpec((RB, D), lambda i: (i, 0))
    wblk = pl.BlockSpec((D, D), lambda i: (0, 0))
    vblk = pl.BlockSpec((1, D), lambda i: (0, 0))
    return pl.pallas_call(
        _pool_body,
        grid=(GRID,),
        in_specs=[blk, blk, blk, blk, wblk, wblk, wblk, wblk, vblk,
                  wblk, vblk, wblk, vblk,
                  pl.BlockSpec((D, 1), lambda i: (0, 0)),
                  pl.BlockSpec((1, 1), lambda i: (0, 0))],
        out_specs=pl.BlockSpec((1, D), lambda i: (0, 0)),
        out_shape=jax.ShapeDtypeStruct((1, D), jnp.float32),
        scratch_shapes=[
            pltpu.SMEM((1, 1), jnp.float32),
            pltpu.SMEM((1, 1), jnp.float32),
            pltpu.VMEM((1, D), jnp.float32),
        ],
    )(h0, h1, h2, h3, wp0, wp1, wp2, wp3, bp, wa, ba, wb, bb, wc, bc)


def kernel(x, edge_index, W_fc, b_fc, conv_W1, conv_b1, conv_ln_g,
           conv_ln_b, conv_W2, conv_b2, conv_t, norm_g, norm_b, W_phi,
           b_phi, Wa, ba, Wb, bb, Wc, bc):
    src = edge_index[0]
    dst = edge_index[1]
    pad = E_PAD - E
    srcp = jnp.concatenate([src, jnp.zeros((pad,), jnp.int32)])
    dstp = jnp.concatenate([dst, jnp.full((pad,), N, jnp.int32)])
    zrows = jnp.zeros((IDX_B, D), jnp.float32)

    h, k = _fc_call(x, W_fc, b_fc.reshape(1, D))
    hs = [h]
    for i in range(3):
        u, v = _prep_call(h, k, conv_t[i].reshape(1, 1))
        nump, denp = _agg_call(u, v, srcp, dstp, zrows)
        h, k = _post_call(
            i == 0, nump, denp, h,
            conv_W1[i], conv_b1[i].reshape(1, 2 * D),
            conv_ln_g[i].reshape(1, 2 * D), conv_ln_b[i].reshape(1, 2 * D),
            conv_W2[i], conv_b2[i].reshape(1, D),
            norm_g[i].reshape(1, D), norm_b[i].reshape(1, D),
        )
        hs.append(h)

    H = _pool_call(
        hs[0], hs[1], hs[2], hs[3],
        W_phi[0:D], W_phi[D:2 * D], W_phi[2 * D:3 * D], W_phi[3 * D:4 * D],
        b_phi.reshape(1, D), Wa, ba.reshape(1, D), Wb, bb.reshape(1, D),
        Wc, bc.reshape(1, 1),
    )
    return H


# speedup vs baseline: 1.5210x; 1.0291x over previous
import functools

import jax
import jax.numpy as jnp
from jax import lax
from jax.experimental import pallas as pl
from jax.experimental.pallas import tpu as pltpu
from jax.experimental.pallas import tpu_sc as plsc

N = 10000
E = 320000
D = 128
EPS = 1e-7
LN_EPS = 1e-5

RB = 1000
GRID = N // RB

SC_TILES = 16
IDX_B = 128
CHUNKS = 158
EDGES_PER_TILE = CHUNKS * IDX_B
E_PAD = SC_TILES * EDGES_PER_TILE
E_ALLOC = E_PAD + IDX_B
NP = 10112
RPT = NP // SC_TILES


def _fc_body(x_ref, w_ref, b_ref, h_ref, k_ref):
    h = jnp.dot(x_ref[...], w_ref[...], preferred_element_type=jnp.float32)
    h = jnp.maximum(h + b_ref[...], 0.0)
    h_ref[...] = h
    m = jnp.max(h, axis=0, keepdims=True) + EPS
    @pl.when(pl.program_id(0) == 0)
    def _():
        k_ref[...] = m
    k_ref[...] = jnp.maximum(k_ref[...], m)


def _fc_call(x, w, b):
    return pl.pallas_call(
        _fc_body,
        grid=(GRID,),
        in_specs=[
            pl.BlockSpec((RB, D), lambda i: (i, 0)),
            pl.BlockSpec((D, D), lambda i: (0, 0)),
            pl.BlockSpec((1, D), lambda i: (0, 0)),
        ],
        out_specs=[
            pl.BlockSpec((RB, D), lambda i: (i, 0)),
            pl.BlockSpec((1, D), lambda i: (0, 0)),
        ],
        out_shape=[
            jax.ShapeDtypeStruct((N, D), jnp.float32),
            jax.ShapeDtypeStruct((1, D), jnp.float32),
        ],
    )(x, w, b)


def _prep_body(h_ref, k_ref, t_ref, u_ref, v_ref):
    r = jnp.maximum(h_ref[...], 0.0) + EPS
    t = t_ref[0, 0]
    w = jnp.exp(t * (r - k_ref[...]))
    u_ref[...] = r * w
    v_ref[...] = w


def _prep_call(h, k, t):
    return pl.pallas_call(
        _prep_body,
        grid=(GRID,),
        in_specs=[
            pl.BlockSpec((RB, D), lambda i: (i, 0)),
            pl.BlockSpec((1, D), lambda i: (0, 0)),
            pl.BlockSpec((1, 1), lambda i: (0, 0)),
        ],
        out_specs=[
            pl.BlockSpec((RB, D), lambda i: (i, 0)),
            pl.BlockSpec((RB, D), lambda i: (i, 0)),
        ],
        out_shape=[
            jax.ShapeDtypeStruct((N, D), jnp.float32),
            jax.ShapeDtypeStruct((N, D), jnp.float32),
        ],
    )(h, k, t)


def _agg_body(u_hbm, v_hbm, src_hbm, dst_hbm, z_hbm,
              num_hbm, den_hbm,
              sidx0, didx0, sidx1, didx1, rows, acc,
              gsem, isem_s, isem_d):
    c = lax.axis_index("c")
    s = lax.axis_index("s")
    r0 = s * RPT

    pltpu.sync_copy(z_hbm, rows)
    for kk in range(4):
        pltpu.sync_copy(rows, acc.at[pl.ds(r0 + kk * IDX_B, IDX_B)])
    pltpu.sync_copy(rows.at[pl.ds(0, RPT - 4 * IDX_B)],
                    acc.at[pl.ds(r0 + 4 * IDX_B, RPT - 4 * IDX_B)])
    plsc.subcore_barrier()

    def run(tab_hbm):
        ebase = s * EDGES_PER_TILE
        pltpu.sync_copy(src_hbm.at[pl.ds(ebase, IDX_B)], sidx0)
        pltpu.sync_copy(dst_hbm.at[pl.ds(ebase, IDX_B)], didx0)

        def outer(jj, carry):
            for b in range(2):
                j = 2 * jj + b
                if b == 0:
                    cur_s, cur_d, nxt_s, nxt_d = sidx0, didx0, sidx1, didx1
                else:
                    cur_s, cur_d, nxt_s, nxt_d = sidx1, didx1, sidx0, didx0
                e1 = ebase + (j + 1) * IDX_B
                dS = pltpu.async_copy(src_hbm.at[pl.ds(e1, IDX_B)],
                                      nxt_s, isem_s)
                dD = pltpu.async_copy(dst_hbm.at[pl.ds(e1, IDX_B)],
                                      nxt_d, isem_d)
                pltpu.async_copy(tab_hbm.at[cur_s], rows, gsem).wait()
                pltpu.sync_copy(rows, acc.at[cur_d], add=True)
                dS.wait()
                dD.wait()
            return carry

        lax.fori_loop(0, CHUNKS // 2, outer, 0)

    @pl.when(c == 0)
    def _():
        run(u_hbm)

    @pl.when(c == 1)
    def _():
        run(v_hbm)

    plsc.subcore_barrier()

    @pl.when(c == 0)
    def _():
        pltpu.sync_copy(acc.at[pl.ds(r0, RPT)], num_hbm.at[pl.ds(r0, RPT)])

    @pl.when(c == 1)
    def _():
        pltpu.sync_copy(acc.at[pl.ds(r0, RPT)], den_hbm.at[pl.ds(r0, RPT)])


def _agg_call(u, v, srcp, dstp, zrows):
    fn = pl.kernel(
        _agg_body,
        out_type=[
            jax.ShapeDtypeStruct((NP, D), jnp.float32),
            jax.ShapeDtypeStruct((NP, D), jnp.float32),
        ],
        mesh=plsc.VectorSubcoreMesh(core_axis_name="c", subcore_axis_name="s"),
        scratch_types=[
            pltpu.VMEM((IDX_B,), jnp.int32),
            pltpu.VMEM((IDX_B,), jnp.int32),
            pltpu.VMEM((IDX_B,), jnp.int32),
            pltpu.VMEM((IDX_B,), jnp.int32),
            pltpu.VMEM((IDX_B, D), jnp.float32),
            pltpu.VMEM_SHARED((NP, D), jnp.float32),
            pltpu.SemaphoreType.DMA,
            pltpu.SemaphoreType.DMA,
            pltpu.SemaphoreType.DMA,
        ],
    )
    return fn(u, v, srcp, dstp, zrows)


def _post_body(first, num_ref, den_ref, h_ref, w1_ref, b1_ref, g1_ref,
               bt1_ref, w2_ref, b2_ref, ng_ref, nb_ref, h_out, k_ref):
    h = h_ref[...]
    agg = num_ref[...] / (den_ref[...] + 1e-16) + h
    hm = jnp.dot(agg, w1_ref[...], preferred_element_type=jnp.float32)
    hm = hm + b1_ref[...]
    mu = jnp.mean(hm, axis=-1, keepdims=True)
    dv = hm - mu
    var = jnp.mean(dv * dv, axis=-1, keepdims=True)
    hm = dv * lax.rsqrt(var + LN_EPS) * g1_ref[...] + bt1_ref[...]
    hm = jnp.maximum(hm, 0.0)
    hc = jnp.dot(hm, w2_ref[...], preferred_element_type=jnp.float32)
    hc = hc + b2_ref[...]
    if first:
        hnew = hc
    else:
        mu2 = jnp.mean(hc, axis=-1, keepdims=True)
        dv2 = hc - mu2
        var2 = jnp.mean(dv2 * dv2, axis=-1, keepdims=True)
        hc = dv2 * lax.rsqrt(var2 + LN_EPS) * ng_ref[...] + nb_ref[...]
        hnew = h + jnp.maximum(hc, 0.0)
    h_out[...] = hnew
    m = jnp.max(jnp.maximum(hnew, 0.0), axis=0, keepdims=True) + EPS
    @pl.when(pl.program_id(0) == 0)
    def _():
        k_ref[...] = m
    k_ref[...] = jnp.maximum(k_ref[...], m)


def _post_call(first, nump, denp, h, w1, b1, g1, bt1, w2, b2, ng, nb):
    return pl.pallas_call(
        functools.partial(_post_body, first),
        grid=(GRID,),
        in_specs=[
            pl.BlockSpec((RB, D), lambda i: (i, 0)),
            pl.BlockSpec((RB, D), lambda i: (i, 0)),
            pl.BlockSpec((RB, D), lambda i: (i, 0)),
            pl.BlockSpec((D, 2 * D), lambda i: (0, 0)),
            pl.BlockSpec((1, 2 * D), lambda i: (0, 0)),
            pl.BlockSpec((1, 2 * D), lambda i: (0, 0)),
            pl.BlockSpec((1, 2 * D), lambda i: (0, 0)),
            pl.BlockSpec((2 * D, D), lambda i: (0, 0)),
            pl.BlockSpec((1, D), lambda i: (0, 0)),
            pl.BlockSpec((1, D), lambda i: (0, 0)),
            pl.BlockSpec((1, D), lambda i: (0, 0)),
        ],
        out_specs=[
            pl.BlockSpec((RB, D), lambda i: (i, 0)),
            pl.BlockSpec((1, D), lambda i: (0, 0)),
        ],
        out_shape=[
            jax.ShapeDtypeStruct((N, D), jnp.float32),
            jax.ShapeDtypeStruct((1, D), jnp.float32),
        ],
    )(nump, denp, h, w1, b1, g1, bt1, w2, b2, ng, nb)


def _pool_body(h0_ref, h1_ref, h2_ref, h3_ref, wp0_ref, wp1_ref, wp2_ref,
               wp3_ref, bp_ref, wa_ref, ba_ref, wb_ref, bb_ref, wc_ref,
               bc_ref, out_ref, m_ref, d_ref, acc_ref):
    hp = jnp.dot(h0_ref[...], wp0_ref[...], preferred_element_type=jnp.float32)
    hp = hp + jnp.dot(h1_ref[...], wp1_ref[...], preferred_element_type=jnp.float32)
    hp = hp + jnp.dot(h2_ref[...], wp2_ref[...], preferred_element_type=jnp.float32)
    hp = hp + jnp.dot(h3_ref[...], wp3_ref[...], preferred_element_type=jnp.float32)
    hp = jnp.maximum(hp + bp_ref[...], 0.0)
    a = jnp.tanh(jnp.dot(hp, wa_ref[...], preferred_element_type=jnp.float32) + ba_ref[...])
    g = jnp.dot(hp, wb_ref[...], preferred_element_type=jnp.float32) + bb_ref[...]
    g = 1.0 / (1.0 + jnp.exp(-g))
    s = jnp.dot(a * g, wc_ref[...], preferred_element_type=jnp.float32) + bc_ref[...]

    @pl.when(pl.program_id(0) == 0)
    def _():
        m_ref[0, 0] = -1e30
        d_ref[0, 0] = 0.0
        acc_ref[...] = jnp.zeros_like(acc_ref)

    bm = jnp.max(s)
    m_old = m_ref[0, 0]
    m_new = jnp.maximum(m_old, bm)
    scale = jnp.exp(m_old - m_new)
    w = jnp.exp(s - m_new)
    d_ref[0, 0] = d_ref[0, 0] * scale + jnp.sum(w)
    acc_ref[...] = acc_ref[...] * scale + jnp.sum(w * hp, axis=0, keepdims=True)
    m_ref[0, 0] = m_new

    @pl.when(pl.program_id(0) == pl.num_programs(0) - 1)
    def _():
        out_ref[...] = acc_ref[...] / d_ref[0, 0]


def _pool_call(h0, h1, h2, h3, wp0, wp1, wp2, wp3, bp, wa, ba, wb, bb, wc, bc):
    blk = pl.BlockSpec((RB, D), lambda i: (i, 0))
    wblk = pl.BlockSpec((D, D), lambda i: (0, 0))
    vblk = pl.BlockSpec((1, D), lambda i: (0, 0))
    return pl.pallas_call(
        _pool_body,
        grid=(GRID,),
        in_specs=[blk, blk, blk, blk, wblk, wblk, wblk, wblk, vblk,
                  wblk, vblk, wblk, vblk,
                  pl.BlockSpec((D, 1), lambda i: (0, 0)),
                  pl.BlockSpec((1, 1), lambda i: (0, 0))],
        out_specs=pl.BlockSpec((1, D), lambda i: (0, 0)),
        out_shape=jax.ShapeDtypeStruct((1, D), jnp.float32),
        scratch_shapes=[
            pltpu.SMEM((1, 1), jnp.float32),
            pltpu.SMEM((1, 1), jnp.float32),
            pltpu.VMEM((1, D), jnp.float32),
        ],
    )(h0, h1, h2, h3, wp0, wp1, wp2, wp3, bp, wa, ba, wb, bb, wc, bc)


def kernel(x, edge_index, W_fc, b_fc, conv_W1, conv_b1, conv_ln_g,
           conv_ln_b, conv_W2, conv_b2, conv_t, norm_g, norm_b, W_phi,
           b_phi, Wa, ba, Wb, bb, Wc, bc):
    src = edge_index[0]
    dst = edge_index[1]
    pad = E_ALLOC - E
    srcp = jnp.concatenate([src, jnp.zeros((pad,), jnp.int32)])
    dstp = jnp.concatenate([dst, jnp.full((pad,), N, jnp.int32)])
    zrows = jnp.zeros((IDX_B, D), jnp.float32)

    h, k = _fc_call(x, W_fc, b_fc.reshape(1, D))
    hs = [h]
    for i in range(3):
        u, v = _prep_call(h, k, conv_t[i].reshape(1, 1))
        nump, denp = _agg_call(u, v, srcp, dstp, zrows)
        h, k = _post_call(
            i == 0, nump, denp, h,
            conv_W1[i], conv_b1[i].reshape(1, 2 * D),
            conv_ln_g[i].reshape(1, 2 * D), conv_ln_b[i].reshape(1, 2 * D),
            conv_W2[i], conv_b2[i].reshape(1, D),
            norm_g[i].reshape(1, D), norm_b[i].reshape(1, D),
        )
        hs.append(h)

    H = _pool_call(
        hs[0], hs[1], hs[2], hs[3],
        W_phi[0:D], W_phi[D:2 * D], W_phi[2 * D:3 * D], W_phi[3 * D:4 * D],
        b_phi.reshape(1, D), Wa, ba.reshape(1, D), Wb, bb.reshape(1, D),
        Wc, bc.reshape(1, 1),
    )
    return H


# speedup vs baseline: 1.7924x; 1.1784x over previous
import functools

import jax
import jax.numpy as jnp
from jax import lax
from jax.experimental import pallas as pl
from jax.experimental.pallas import tpu as pltpu
from jax.experimental.pallas import tpu_sc as plsc

N = 10000
E = 320000
D = 128
EPS = 1e-7
LN_EPS = 1e-5

RB = 1000
GRID = N // RB

SC_TILES = 16
IDX_B = 128
CHUNKS = 158
EDGES_PER_TILE = CHUNKS * IDX_B
E_PAD = SC_TILES * EDGES_PER_TILE
E_ALLOC = E_PAD + IDX_B
NP = 10112
RPT = NP // SC_TILES


def _fc_body(x_ref, w_ref, b_ref, h_ref, k_ref):
    h = jnp.dot(x_ref[...], w_ref[...], preferred_element_type=jnp.float32)
    h = jnp.maximum(h + b_ref[...], 0.0)
    h_ref[...] = h
    m = jnp.max(h, axis=0, keepdims=True) + EPS
    @pl.when(pl.program_id(0) == 0)
    def _():
        k_ref[...] = m
    k_ref[...] = jnp.maximum(k_ref[...], m)


def _fc_call(x, w, b):
    return pl.pallas_call(
        _fc_body,
        grid=(GRID,),
        in_specs=[
            pl.BlockSpec((RB, D), lambda i: (i, 0)),
            pl.BlockSpec((D, D), lambda i: (0, 0)),
            pl.BlockSpec((1, D), lambda i: (0, 0)),
        ],
        out_specs=[
            pl.BlockSpec((RB, D), lambda i: (i, 0)),
            pl.BlockSpec((1, D), lambda i: (0, 0)),
        ],
        out_shape=[
            jax.ShapeDtypeStruct((N, D), jnp.float32),
            jax.ShapeDtypeStruct((1, D), jnp.float32),
        ],
    )(x, w, b)


def _prep_body(h_ref, k_ref, t_ref, u_ref, v_ref):
    r = jnp.maximum(h_ref[...], 0.0) + EPS
    t = t_ref[0, 0]
    w = jnp.exp(t * (r - k_ref[...]))
    u_ref[...] = r * w
    v_ref[...] = w


def _prep_call(h, k, t):
    return pl.pallas_call(
        _prep_body,
        grid=(GRID,),
        in_specs=[
            pl.BlockSpec((RB, D), lambda i: (i, 0)),
            pl.BlockSpec((1, D), lambda i: (0, 0)),
            pl.BlockSpec((1, 1), lambda i: (0, 0)),
        ],
        out_specs=[
            pl.BlockSpec((RB, D), lambda i: (i, 0)),
            pl.BlockSpec((RB, D), lambda i: (i, 0)),
        ],
        out_shape=[
            jax.ShapeDtypeStruct((N, D), jnp.float32),
            jax.ShapeDtypeStruct((N, D), jnp.float32),
        ],
    )(h, k, t)


def _agg_body(u_hbm, v_hbm, src_hbm, dst_hbm, z_hbm,
              num_hbm, den_hbm,
              sidx0, didx0, sidx1, didx1, rows, rows1, acc,
              gsem, isem_s, isem_d, ssem0, ssem1):
    c = lax.axis_index("c")
    s = lax.axis_index("s")
    r0 = s * RPT

    pltpu.sync_copy(z_hbm, rows)
    pltpu.sync_copy(z_hbm, rows1)
    for kk in range(4):
        pltpu.sync_copy(rows, acc.at[pl.ds(r0 + kk * IDX_B, IDX_B)])
    pltpu.sync_copy(rows.at[pl.ds(0, RPT - 4 * IDX_B)],
                    acc.at[pl.ds(r0 + 4 * IDX_B, RPT - 4 * IDX_B)])
    plsc.subcore_barrier()

    def run(tab_hbm):
        ebase = s * EDGES_PER_TILE
        pltpu.sync_copy(src_hbm.at[pl.ds(ebase, IDX_B)], sidx0)
        pltpu.sync_copy(dst_hbm.at[pl.ds(ebase, IDX_B)], didx0)
        pltpu.async_copy(rows, acc.at[didx0], ssem0, add=True)
        pltpu.async_copy(rows1, acc.at[didx0], ssem1, add=True)

        def outer(jj, carry):
            for b in range(2):
                j = 2 * jj + b
                if b == 0:
                    cur_s, cur_d, nxt_s, nxt_d = sidx0, didx0, sidx1, didx1
                    rb, sb = rows, ssem0
                else:
                    cur_s, cur_d, nxt_s, nxt_d = sidx1, didx1, sidx0, didx0
                    rb, sb = rows1, ssem1
                e1 = ebase + (j + 1) * IDX_B
                dS = pltpu.async_copy(src_hbm.at[pl.ds(e1, IDX_B)],
                                      nxt_s, isem_s)
                dD = pltpu.async_copy(dst_hbm.at[pl.ds(e1, IDX_B)],
                                      nxt_d, isem_d)
                pltpu.make_async_copy(rb, acc.at[cur_d], sb).wait()
                pltpu.async_copy(tab_hbm.at[cur_s], rb, gsem).wait()
                pltpu.async_copy(rb, acc.at[cur_d], sb, add=True)
                dS.wait()
                dD.wait()
            return carry

        lax.fori_loop(0, CHUNKS // 2, outer, 0)
        pltpu.make_async_copy(rows, acc.at[didx0], ssem0).wait()
        pltpu.make_async_copy(rows1, acc.at[didx0], ssem1).wait()

    @pl.when(c == 0)
    def _():
        run(u_hbm)

    @pl.when(c == 1)
    def _():
        run(v_hbm)

    plsc.subcore_barrier()

    @pl.when(c == 0)
    def _():
        pltpu.sync_copy(acc.at[pl.ds(r0, RPT)], num_hbm.at[pl.ds(r0, RPT)])

    @pl.when(c == 1)
    def _():
        pltpu.sync_copy(acc.at[pl.ds(r0, RPT)], den_hbm.at[pl.ds(r0, RPT)])


def _agg_call(u, v, srcp, dstp, zrows):
    fn = pl.kernel(
        _agg_body,
        out_type=[
            jax.ShapeDtypeStruct((NP, D), jnp.float32),
            jax.ShapeDtypeStruct((NP, D), jnp.float32),
        ],
        mesh=plsc.VectorSubcoreMesh(core_axis_name="c", subcore_axis_name="s"),
        scratch_types=[
            pltpu.VMEM((IDX_B,), jnp.int32),
            pltpu.VMEM((IDX_B,), jnp.int32),
            pltpu.VMEM((IDX_B,), jnp.int32),
            pltpu.VMEM((IDX_B,), jnp.int32),
            pltpu.VMEM((IDX_B, D), jnp.float32),
            pltpu.VMEM((IDX_B, D), jnp.float32),
            pltpu.VMEM_SHARED((NP, D), jnp.float32),
            pltpu.SemaphoreType.DMA,
            pltpu.SemaphoreType.DMA,
            pltpu.SemaphoreType.DMA,
            pltpu.SemaphoreType.DMA,
            pltpu.SemaphoreType.DMA,
        ],
    )
    return fn(u, v, srcp, dstp, zrows)


def _post_body(first, num_ref, den_ref, h_ref, w1_ref, b1_ref, g1_ref,
               bt1_ref, w2_ref, b2_ref, ng_ref, nb_ref, h_out, k_ref):
    h = h_ref[...]
    agg = num_ref[...] / (den_ref[...] + 1e-16) + h
    hm = jnp.dot(agg, w1_ref[...], preferred_element_type=jnp.float32)
    hm = hm + b1_ref[...]
    mu = jnp.mean(hm, axis=-1, keepdims=True)
    dv = hm - mu
    var = jnp.mean(dv * dv, axis=-1, keepdims=True)
    hm = dv * lax.rsqrt(var + LN_EPS) * g1_ref[...] + bt1_ref[...]
    hm = jnp.maximum(hm, 0.0)
    hc = jnp.dot(hm, w2_ref[...], preferred_element_type=jnp.float32)
    hc = hc + b2_ref[...]
    if first:
        hnew = hc
    else:
        mu2 = jnp.mean(hc, axis=-1, keepdims=True)
        dv2 = hc - mu2
        var2 = jnp.mean(dv2 * dv2, axis=-1, keepdims=True)
        hc = dv2 * lax.rsqrt(var2 + LN_EPS) * ng_ref[...] + nb_ref[...]
        hnew = h + jnp.maximum(hc, 0.0)
    h_out[...] = hnew
    m = jnp.max(jnp.maximum(hnew, 0.0), axis=0, keepdims=True) + EPS
    @pl.when(pl.program_id(0) == 0)
    def _():
        k_ref[...] = m
    k_ref[...] = jnp.maximum(k_ref[...], m)


def _post_call(first, nump, denp, h, w1, b1, g1, bt1, w2, b2, ng, nb):
    return pl.pallas_call(
        functools.partial(_post_body, first),
        grid=(GRID,),
        in_specs=[
            pl.BlockSpec((RB, D), lambda i: (i, 0)),
            pl.BlockSpec((RB, D), lambda i: (i, 0)),
            pl.BlockSpec((RB, D), lambda i: (i, 0)),
            pl.BlockSpec((D, 2 * D), lambda i: (0, 0)),
            pl.BlockSpec((1, 2 * D), lambda i: (0, 0)),
            pl.BlockSpec((1, 2 * D), lambda i: (0, 0)),
            pl.BlockSpec((1, 2 * D), lambda i: (0, 0)),
            pl.BlockSpec((2 * D, D), lambda i: (0, 0)),
            pl.BlockSpec((1, D), lambda i: (0, 0)),
            pl.BlockSpec((1, D), lambda i: (0, 0)),
            pl.BlockSpec((1, D), lambda i: (0, 0)),
        ],
        out_specs=[
            pl.BlockSpec((RB, D), lambda i: (i, 0)),
            pl.BlockSpec((1, D), lambda i: (0, 0)),
        ],
        out_shape=[
            jax.ShapeDtypeStruct((N, D), jnp.float32),
            jax.ShapeDtypeStruct((1, D), jnp.float32),
        ],
    )(nump, denp, h, w1, b1, g1, bt1, w2, b2, ng, nb)


def _pool_body(h0_ref, h1_ref, h2_ref, h3_ref, wp0_ref, wp1_ref, wp2_ref,
               wp3_ref, bp_ref, wa_ref, ba_ref, wb_ref, bb_ref, wc_ref,
               bc_ref, out_ref, m_ref, d_ref, acc_ref):
    hp = jnp.dot(h0_ref[...], wp0_ref[...], preferred_element_type=jnp.float32)
    hp = hp + jnp.dot(h1_ref[...], wp1_ref[...], preferred_element_type=jnp.float32)
    hp = hp + jnp.dot(h2_ref[...], wp2_ref[...], preferred_element_type=jnp.float32)
    hp = hp + jnp.dot(h3_ref[...], wp3_ref[...], preferred_element_type=jnp.float32)
    hp = jnp.maximum(hp + bp_ref[...], 0.0)
    a = jnp.tanh(jnp.dot(hp, wa_ref[...], preferred_element_type=jnp.float32) + ba_ref[...])
    g = jnp.dot(hp, wb_ref[...], preferred_element_type=jnp.float32) + bb_ref[...]
    g = 1.0 / (1.0 + jnp.exp(-g))
    s = jnp.dot(a * g, wc_ref[...], preferred_element_type=jnp.float32) + bc_ref[...]

    @pl.when(pl.program_id(0) == 0)
    def _():
        m_ref[0, 0] = -1e30
        d_ref[0, 0] = 0.0
        acc_ref[...] = jnp.zeros_like(acc_ref)

    bm = jnp.max(s)
    m_old = m_ref[0, 0]
    m_new = jnp.maximum(m_old, bm)
    scale = jnp.exp(m_old - m_new)
    w = jnp.exp(s - m_new)
    d_ref[0, 0] = d_ref[0, 0] * scale + jnp.sum(w)
    acc_ref[...] = acc_ref[...] * scale + jnp.sum(w * hp, axis=0, keepdims=True)
    m_ref[0, 0] = m_new

    @pl.when(pl.program_id(0) == pl.num_programs(0) - 1)
    def _():
        out_ref[...] = acc_ref[...] / d_ref[0, 0]


def _pool_call(h0, h1, h2, h3, wp0, wp1, wp2, wp3, bp, wa, ba, wb, bb, wc, bc):
    blk = pl.BlockSpec((RB, D), lambda i: (i, 0))
    wblk = pl.BlockSpec((D, D), lambda i: (0, 0))
    vblk = pl.BlockSpec((1, D), lambda i: (0, 0))
    return pl.pallas_call(
        _pool_body,
        grid=(GRID,),
        in_specs=[blk, blk, blk, blk, wblk, wblk, wblk, wblk, vblk,
                  wblk, vblk, wblk, vblk,
                  pl.BlockSpec((D, 1), lambda i: (0, 0)),
                  pl.BlockSpec((1, 1), lambda i: (0, 0))],
        out_specs=pl.BlockSpec((1, D), lambda i: (0, 0)),
        out_shape=jax.ShapeDtypeStruct((1, D), jnp.float32),
        scratch_shapes=[
            pltpu.SMEM((1, 1), jnp.float32),
            pltpu.SMEM((1, 1), jnp.float32),
            pltpu.VMEM((1, D), jnp.float32),
        ],
    )(h0, h1, h2, h3, wp0, wp1, wp2, wp3, bp, wa, ba, wb, bb, wc, bc)


def kernel(x, edge_index, W_fc, b_fc, conv_W1, conv_b1, conv_ln_g,
           conv_ln_b, conv_W2, conv_b2, conv_t, norm_g, norm_b, W_phi,
           b_phi, Wa, ba, Wb, bb, Wc, bc):
    src = edge_index[0]
    dst = edge_index[1]
    pad = E_ALLOC - E
    srcp = jnp.concatenate([src, jnp.zeros((pad,), jnp.int32)])
    dstp = jnp.concatenate([dst, jnp.full((pad,), N, jnp.int32)])
    zrows = jnp.zeros((IDX_B, D), jnp.float32)

    h, k = _fc_call(x, W_fc, b_fc.reshape(1, D))
    hs = [h]
    for i in range(3):
        u, v = _prep_call(h, k, conv_t[i].reshape(1, 1))
        nump, denp = _agg_call(u, v, srcp, dstp, zrows)
        h, k = _post_call(
            i == 0, nump, denp, h,
            conv_W1[i], conv_b1[i].reshape(1, 2 * D),
            conv_ln_g[i].reshape(1, 2 * D), conv_ln_b[i].reshape(1, 2 * D),
            conv_W2[i], conv_b2[i].reshape(1, D),
            norm_g[i].reshape(1, D), norm_b[i].reshape(1, D),
        )
        hs.append(h)

    H = _pool_call(
        hs[0], hs[1], hs[2], hs[3],
        W_phi[0:D], W_phi[D:2 * D], W_phi[2 * D:3 * D], W_phi[3 * D:4 * D],
        b_phi.reshape(1, D), Wa, ba.reshape(1, D), Wb, bb.reshape(1, D),
        Wc, bc.reshape(1, 1),
    )
    return H
